# Initial kernel scaffold; baseline (speedup 1.0000x reference)
import jax
import jax.numpy as jnp
from jax.experimental import pallas as pl


def kernel(x, t, emb_table, fc_table, fc_bias, lin_w, lin_b):
    raise NotImplementedError("write your pallas kernel here")


# speedup vs baseline: 1.4538x; 1.4538x over previous
import functools

import jax
import jax.numpy as jnp
import numpy as np
from jax import lax
from jax.experimental import pallas as pl
from jax.experimental.pallas import tpu as pltpu
from jax.experimental.pallas import tpu_sc as plsc

_FIELD_DIMS = np.full(27, 40000, dtype=np.int64)
_USED = _FIELD_DIMS[:-1]
_NF = len(_USED)
_D = 16
_VOCAB = int(_USED.sum())
_OFFSETS = np.concatenate([[0], np.cumsum(_USED)[:-1]]).astype(np.int32)
_B = 16384

_NW = 32
_BPW = _B // _NW
_C = 64
_NCHUNK = _BPW // _C
_CI = _C * _NF


def _sc_body(idxb_hbm, idxf_hbm, t_hbm, emb_hbm, fc_hbm, lw_hbm, bias_hbm,
             out_hbm,
             idxb_v, idxf_v, rows_v, fc_v, t_v, lw_v, bias_v, oo_v,
             sem_e, sem_f):
    nc = 2
    wid = lax.axis_index("s") * nc + lax.axis_index("c")

    pltpu.sync_copy(lw_hbm, lw_v)
    pltpu.sync_copy(bias_hbm, bias_v)
    lwv = lw_v[0, :]
    bv = bias_v[0, :]

    for i in range(_NCHUNK):
        b0 = wid * _BPW + i * _C
        i0 = b0 * _NF
        pltpu.sync_copy(idxb_hbm.at[pl.ds(i0, _CI)], idxb_v)
        pltpu.sync_copy(idxf_hbm.at[pl.ds(i0, _CI)], idxf_v)
        pltpu.sync_copy(t_hbm.at[pl.ds(b0, _C), :], t_v)
        cp_e = pltpu.async_copy(emb_hbm.at[idxb_v], rows_v, sem_e)
        cp_f = pltpu.async_copy(fc_hbm.at[idxf_v], fc_v, sem_f)
        cp_e.wait()
        cp_f.wait()

        lanes = lax.iota(jnp.int32, 16)
        for g in range(_C // 16):
            def row_body(j, acc):
                c = g * 16 + j
                tb = t_v[c, :]
                s = tb
                q = tb * tb
                base = c * _NF
                for f in range(_NF):
                    v = rows_v[base + f, :]
                    s = s + v
                    q = q + v * v
                w = 0.5 * (s * s - q) + tb * lwv
                r = jnp.sum(w)
                return jnp.where(lanes == j, r, acc)

            fm = lax.fori_loop(0, 16, row_body, jnp.zeros((16,), jnp.float32))
            acc = fc_v[pl.ds(g * 16, 16)]
            for f in range(1, _NF):
                acc = acc + fc_v[pl.ds(f * _C + g * 16, 16)]
            oo_v[pl.ds(g * 16, 16)] = fm + acc + bv

        pltpu.sync_copy(oo_v, out_hbm.at[pl.ds(b0, _C)])


@functools.partial(jax.jit, static_argnames=())
def kernel(x, t, emb_table, fc_table, fc_bias, lin_w, lin_b):
    xi = x + jnp.asarray(_OFFSETS, dtype=x.dtype)[None, :]
    idx_b = xi.reshape(_B * _NF)
    idx_f = (xi.reshape(_B // _C, _C, _NF)
               .transpose(0, 2, 1)
               .reshape(_B * _NF))
    fc_flat = fc_table.reshape(_VOCAB)
    bias_vec = jnp.broadcast_to((fc_bias + lin_b).reshape(1, 1), (1, _D))

    mesh = plsc.VectorSubcoreMesh(core_axis_name="c", subcore_axis_name="s")
    fn = pl.kernel(
        _sc_body,
        out_type=jax.ShapeDtypeStruct((_B,), jnp.float32),
        mesh=mesh,
        compiler_params=pltpu.CompilerParams(
            needs_layout_passes=False, use_tc_tiling_on_sc=False),
        scratch_types=[
            pltpu.VMEM((_CI,), jnp.int32),
            pltpu.VMEM((_CI,), jnp.int32),
            pltpu.VMEM((_CI, _D), jnp.float32),
            pltpu.VMEM((_CI,), jnp.float32),
            pltpu.VMEM((_C, _D), jnp.float32),
            pltpu.VMEM((1, _D), jnp.float32),
            pltpu.VMEM((1, _D), jnp.float32),
            pltpu.VMEM((_C,), jnp.float32),
            pltpu.SemaphoreType.DMA,
            pltpu.SemaphoreType.DMA,
        ],
    )
    return fn(idx_b, idx_f, t, emb_table, fc_flat, lin_w, bias_vec)


# speedup vs baseline: 1.7091x; 1.1756x over previous
import functools

import jax
import jax.numpy as jnp
import numpy as np
from jax import lax
from jax.experimental import pallas as pl
from jax.experimental.pallas import tpu as pltpu
from jax.experimental.pallas import tpu_sc as plsc

_FIELD_DIMS = np.full(27, 40000, dtype=np.int64)
_USED = _FIELD_DIMS[:-1]
_NF = len(_USED)
_D = 16
_VOCAB = int(_USED.sum())
_OFFSETS = np.concatenate([[0], np.cumsum(_USED)[:-1]]).astype(np.int32)
_B = 16384

_NW = 32
_BPW = _B // _NW
_C = 64
_NCHUNK = _BPW // _C
_CI = _C * _NF

_VB = _VOCAB // 128
_CB = 125


_NCB = (_VB + _NW - 1) // _NW


def _sc_transpose_body(v4_hbm, out_hbm, vin, vout, sem):
    nc = 2
    wid = lax.axis_index("s") * nc + lax.axis_index("c")
    lanes = lax.iota(jnp.int32, 16)
    rowvecs = [w0 * 16 + lanes for w0 in range(8)]
    colvecs = [jnp.full((16,), d, jnp.int32) for d in range(16)]

    def step(k, _):
        c = wid * _NCB + k

        @pl.when(c < _VB)
        def _():
            pltpu.sync_copy(v4_hbm.at[:, c, :, :], vin)
            for d in range(16):
                for w0 in range(8):
                    vec = vin[d // 8, d % 8, pl.ds(w0 * 16, 16)]
                    plsc.store_scatter(vout, [rowvecs[w0], colvecs[d]], vec)
            pltpu.sync_copy(vout, out_hbm.at[pl.ds(c * 128, 128), :])

        return 0

    lax.fori_loop(0, _NCB, step, 0)


def _sc_body(idx_hbm, t_hbm, emb_hbm, fc_hbm, lw_hbm, bias_hbm,
             out_hbm,
             idx_v, rows_v, fc_v, t_v, lw_v, bias_v, oo_v,
             sem_e, sem_f):
    nc = 2
    wid = lax.axis_index("s") * nc + lax.axis_index("c")

    pltpu.sync_copy(lw_hbm, lw_v)
    pltpu.sync_copy(bias_hbm, bias_v)
    lwv = lw_v[0, :]
    bv = bias_v[0, :]
    lanes = lax.iota(jnp.int32, 16)

    for i in range(_NCHUNK):
        b0 = wid * _BPW + i * _C
        i0 = b0 * _NF
        pltpu.sync_copy(idx_hbm.at[pl.ds(i0, _CI)], idx_v)
        pltpu.sync_copy(t_hbm.at[pl.ds(b0, _C), :], t_v)
        cp_e = pltpu.async_copy(emb_hbm.at[idx_v], rows_v, sem_e)
        cp_f = pltpu.async_copy(fc_hbm.at[idx_v], fc_v, sem_f)
        cp_e.wait()
        cp_f.wait()

        for g in range(_C // 16):
            def row_body(j, acc):
                c = g * 16 + j
                tb = t_v[c, :]
                s = tb
                q = tb * tb
                base = c * _NF
                for f in range(_NF):
                    v = rows_v[base + f, :]
                    s = s + v
                    q = q + v * v
                w = 0.5 * (s * s - q) + tb * lwv
                r = jnp.sum(w)
                return jnp.where(lanes == j, r, acc)

            fm = lax.fori_loop(0, 16, row_body, jnp.zeros((16,), jnp.float32))
            fbase = (g * 16 + lanes) * _NF
            fca = plsc.load_gather(fc_v, [fbase])
            for f in range(1, _NF):
                fca = fca + plsc.load_gather(fc_v, [fbase + f])
            oo_v[pl.ds(g * 16, 16)] = fm + fca + bv

        pltpu.sync_copy(oo_v, out_hbm.at[pl.ds(b0, _C)])


@functools.partial(jax.jit, static_argnames=())
def kernel(x, t, emb_table, fc_table, fc_bias, lin_w, lin_b):
    v4 = emb_table.reshape(_VB, 128, 2, 8).transpose(2, 0, 3, 1)
    mesh_t = plsc.VectorSubcoreMesh(core_axis_name="c", subcore_axis_name="s")
    emb_rm = pl.kernel(
        _sc_transpose_body,
        out_type=jax.ShapeDtypeStruct((_VOCAB, _D), jnp.float32),
        mesh=mesh_t,
        compiler_params=pltpu.CompilerParams(
            needs_layout_passes=False, use_tc_tiling_on_sc=False),
        scratch_types=[
            pltpu.VMEM((2, 8, 128), jnp.float32),
            pltpu.VMEM((128, 16), jnp.float32),
            pltpu.SemaphoreType.DMA,
        ],
    )(v4)

    xi = x + jnp.asarray(_OFFSETS, dtype=x.dtype)[None, :]
    idx_b = xi.reshape(_B * _NF)
    bias_vec = jnp.broadcast_to((fc_bias + lin_b).reshape(1, 1), (1, _D))

    mesh = plsc.VectorSubcoreMesh(core_axis_name="c", subcore_axis_name="s")
    fn = pl.kernel(
        _sc_body,
        out_type=jax.ShapeDtypeStruct((_B,), jnp.float32),
        mesh=mesh,
        compiler_params=pltpu.CompilerParams(
            needs_layout_passes=False, use_tc_tiling_on_sc=False),
        scratch_types=[
            pltpu.VMEM((_CI,), jnp.int32),
            pltpu.VMEM((_CI, _D), jnp.float32),
            pltpu.VMEM((_CI,), jnp.float32),
            pltpu.VMEM((_C, _D), jnp.float32),
            pltpu.VMEM((1, _D), jnp.float32),
            pltpu.VMEM((1, _D), jnp.float32),
            pltpu.VMEM((_C,), jnp.float32),
            pltpu.SemaphoreType.DMA,
            pltpu.SemaphoreType.DMA,
        ],
    )
    return fn(idx_b, t, emb_rm, jnp.squeeze(fc_table, 1), lin_w, bias_vec)


# speedup vs baseline: 3.2174x; 1.8825x over previous
import functools

import jax
import jax.numpy as jnp
import numpy as np
from jax import lax
from jax.experimental import pallas as pl
from jax.experimental.pallas import tpu as pltpu
from jax.experimental.pallas import tpu_sc as plsc

_FIELD_DIMS = np.full(27, 40000, dtype=np.int64)
_USED = _FIELD_DIMS[:-1]
_NF = len(_USED)
_D = 16
_VOCAB = int(_USED.sum())
_OFFSETS = np.concatenate([[0], np.cumsum(_USED)[:-1]]).astype(np.int32)
_B = 16384

_NW = 32
_BPW = _B // _NW
_C = 64
_NCHUNK = _BPW // _C
_CI = _C * _NF

_VB = _VOCAB // 128
_CB = 125


_NCB = (_VB + _NW - 1) // _NW
_CPS = 8
_NCHT = _NCB // _CPS
_ROWS = _CPS * 128


def _sc_transpose_body(v4_hbm, out_hbm,
                       vin0, vin1, vout0, vout1,
                       si0, si1, so0, so1):
    nc = 2
    wid = lax.axis_index("s") * nc + lax.axis_index("c")
    wc0 = wid * _NCB
    lanes = lax.iota(jnp.int32, 16)
    colvecs = [jnp.full((16,), d, jnp.int32) for d in range(16)]
    vins = (vin0, vin1)
    vouts = (vout0, vout1)
    sis = (si0, si1)
    sos = (so0, so1)
    nch = _NCHT + 1

    def c0_of(j):
        return jnp.minimum(wc0 + j * _CPS, _VB - _CPS)

    pltpu.async_copy(v4_hbm.at[:, pl.ds(c0_of(0), _CPS), :, :], vin0, si0)
    pltpu.async_copy(v4_hbm.at[:, pl.ds(c0_of(1), _CPS), :, :], vin1, si1)

    def step(k, _):
        for par in range(2):
            j = 2 * k + par
            c0 = c0_of(j)
            vin = vins[par]
            vout = vouts[par]
            pltpu.make_async_copy(
                v4_hbm.at[:, pl.ds(c0, _CPS), :, :], vin, sis[par]).wait()

            @pl.when(k > 0)
            def _():
                pltpu.make_async_copy(
                    vout, out_hbm.at[pl.ds(c0 * 128, _ROWS), :],
                    sos[par]).wait()

            for cc in range(_CPS):
                for w0 in range(8):
                    rows = lanes + (cc * 128 + w0 * 16)
                    for d in range(16):
                        vec = vin[d // 8, cc, d % 8, pl.ds(w0 * 16, 16)]
                        plsc.store_scatter(vout, [rows, colvecs[d]], vec)

            pltpu.async_copy(
                vout, out_hbm.at[pl.ds(c0 * 128, _ROWS), :], sos[par])

            @pl.when(j + 2 < nch)
            def _():
                pltpu.async_copy(
                    v4_hbm.at[:, pl.ds(c0_of(j + 2), _CPS), :, :],
                    vin, sis[par])

        return 0

    lax.fori_loop(0, nch // 2, step, 0)
    pltpu.make_async_copy(
        vout0, out_hbm.at[pl.ds(0, _ROWS), :], so0).wait()
    pltpu.make_async_copy(
        vout1, out_hbm.at[pl.ds(0, _ROWS), :], so1).wait()


def _sc_body(idx_hbm, t_hbm, emb_hbm, fc_hbm, lw_hbm, bias_hbm,
             out_hbm,
             idx_v, rows_v, fc_v, t_v, lw_v, bias_v, oo_v,
             sem_e, sem_f):
    nc = 2
    wid = lax.axis_index("s") * nc + lax.axis_index("c")

    pltpu.sync_copy(lw_hbm, lw_v)
    pltpu.sync_copy(bias_hbm, bias_v)
    lwv = lw_v[0, :]
    bv = bias_v[0, :]
    lanes = lax.iota(jnp.int32, 16)

    for i in range(_NCHUNK):
        b0 = wid * _BPW + i * _C
        i0 = b0 * _NF
        pltpu.sync_copy(idx_hbm.at[pl.ds(i0, _CI)], idx_v)
        pltpu.sync_copy(t_hbm.at[pl.ds(b0, _C), :], t_v)
        cp_e = pltpu.async_copy(emb_hbm.at[idx_v], rows_v, sem_e)
        cp_f = pltpu.async_copy(fc_hbm.at[idx_v], fc_v, sem_f)
        cp_e.wait()
        cp_f.wait()

        for g in range(_C // 16):
            def row_body(j, acc):
                c = g * 16 + j
                tb = t_v[c, :]
                s = tb
                q = tb * tb
                base = c * _NF
                for f in range(_NF):
                    v = rows_v[base + f, :]
                    s = s + v
                    q = q + v * v
                w = 0.5 * (s * s - q) + tb * lwv
                r = jnp.sum(w)
                return jnp.where(lanes == j, r, acc)

            fm = lax.fori_loop(0, 16, row_body, jnp.zeros((16,), jnp.float32))
            fbase = (g * 16 + lanes) * _NF
            fca = plsc.load_gather(fc_v, [fbase])
            for f in range(1, _NF):
                fca = fca + plsc.load_gather(fc_v, [fbase + f])
            oo_v[pl.ds(g * 16, 16)] = fm + fca + bv

        pltpu.sync_copy(oo_v, out_hbm.at[pl.ds(b0, _C)])


@functools.partial(jax.jit, static_argnames=())
def kernel(x, t, emb_table, fc_table, fc_bias, lin_w, lin_b):
    v4 = emb_table.reshape(_VB, 128, 2, 8).transpose(2, 0, 3, 1)
    mesh_t = plsc.VectorSubcoreMesh(core_axis_name="c", subcore_axis_name="s")
    emb_rm = pl.kernel(
        _sc_transpose_body,
        out_type=jax.ShapeDtypeStruct((_VOCAB, _D), jnp.float32),
        mesh=mesh_t,
        compiler_params=pltpu.CompilerParams(
            needs_layout_passes=False, use_tc_tiling_on_sc=False),
        scratch_types=[
            pltpu.VMEM((2, _CPS, 8, 128), jnp.float32),
            pltpu.VMEM((2, _CPS, 8, 128), jnp.float32),
            pltpu.VMEM((_ROWS, _D), jnp.float32),
            pltpu.VMEM((_ROWS, _D), jnp.float32),
            pltpu.SemaphoreType.DMA,
            pltpu.SemaphoreType.DMA,
            pltpu.SemaphoreType.DMA,
            pltpu.SemaphoreType.DMA,
        ],
    )(v4)

    xi = x + jnp.asarray(_OFFSETS, dtype=x.dtype)[None, :]
    idx_b = xi.reshape(_B * _NF)
    bias_vec = jnp.broadcast_to((fc_bias + lin_b).reshape(1, 1), (1, _D))

    mesh = plsc.VectorSubcoreMesh(core_axis_name="c", subcore_axis_name="s")
    fn = pl.kernel(
        _sc_body,
        out_type=jax.ShapeDtypeStruct((_B,), jnp.float32),
        mesh=mesh,
        compiler_params=pltpu.CompilerParams(
            needs_layout_passes=False, use_tc_tiling_on_sc=False),
        scratch_types=[
            pltpu.VMEM((_CI,), jnp.int32),
            pltpu.VMEM((_CI, _D), jnp.float32),
            pltpu.VMEM((_CI,), jnp.float32),
            pltpu.VMEM((_C, _D), jnp.float32),
            pltpu.VMEM((1, _D), jnp.float32),
            pltpu.VMEM((1, _D), jnp.float32),
            pltpu.VMEM((_C,), jnp.float32),
            pltpu.SemaphoreType.DMA,
            pltpu.SemaphoreType.DMA,
        ],
    )
    return fn(idx_b, t, emb_rm, jnp.squeeze(fc_table, 1), lin_w, bias_vec)


# speedup vs baseline: 3.8325x; 1.1912x over previous
import functools

import jax
import jax.numpy as jnp
import numpy as np
from jax import lax
from jax.experimental import pallas as pl
from jax.experimental.pallas import tpu as pltpu
from jax.experimental.pallas import tpu_sc as plsc

_FIELD_DIMS = np.full(27, 40000, dtype=np.int64)
_USED = _FIELD_DIMS[:-1]
_NF = len(_USED)
_D = 16
_VOCAB = int(_USED.sum())
_OFFSETS = np.concatenate([[0], np.cumsum(_USED)[:-1]]).astype(np.int32)
_B = 16384

_NW = 32
_BPW = _B // _NW
_C = 64
_NCHUNK = _BPW // _C
_CI = _C * _NF

_VB = _VOCAB // 128
_CB = 125


_NCB = (_VB + _NW - 1) // _NW
_CPS = 8
_NCHT = _NCB // _CPS
_ROWS = _CPS * 128
_ROWD = 128 * _D


def _sc_transpose_body(v4_hbm, out_hbm,
                       vin0, vin1, vout0, vout1,
                       si0, si1, so0, so1):
    nc = 2
    wid = lax.axis_index("s") * nc + lax.axis_index("c")
    wc0 = wid * _NCB
    lanes16 = lax.iota(jnp.int32, 16) * 16
    idxs = [lanes16 + d for d in range(16)]
    vins = (vin0, vin1)
    vouts = (vout0, vout1)
    sis = (si0, si1)
    sos = (so0, so1)
    nch = _NCHT + 1

    def c0_of(j):
        return jnp.minimum(wc0 + j * _CPS, _VB - _CPS)

    pltpu.async_copy(v4_hbm.at[:, pl.ds(c0_of(0), _CPS), :, :], vin0, si0)
    pltpu.async_copy(v4_hbm.at[:, pl.ds(c0_of(1), _CPS), :, :], vin1, si1)

    def step(k, _):
        for par in range(2):
            j = 2 * k + par
            c0 = c0_of(j)
            vin = vins[par]
            vout = vouts[par]
            pltpu.make_async_copy(
                v4_hbm.at[:, pl.ds(c0, _CPS), :, :], vin, sis[par]).wait()

            @pl.when(k > 0)
            def _():
                pltpu.make_async_copy(
                    vout.at[pl.ds(0, _ROWD * _CPS)],
                    out_hbm.at[pl.ds(c0 * _ROWD, _ROWD * _CPS)],
                    sos[par]).wait()

            for cc in range(_CPS):
                for w0 in range(8):
                    vecs = [vin[d // 8, cc, d % 8, pl.ds(w0 * 16, 16)]
                            for d in range(16)]
                    off8 = cc * 2048 + w0 * 256
                    for d in range(16):
                        plsc.store_scatter(vout.at[pl.ds(off8, 256)],
                                           [idxs[d]], vecs[d])

            pltpu.async_copy(
                vout.at[pl.ds(0, _ROWD * _CPS)],
                out_hbm.at[pl.ds(c0 * _ROWD, _ROWD * _CPS)], sos[par])

            @pl.when(j + 2 < nch)
            def _():
                pltpu.async_copy(
                    v4_hbm.at[:, pl.ds(c0_of(j + 2), _CPS), :, :],
                    vin, sis[par])

        return 0

    lax.fori_loop(0, nch // 2, step, 0)
    pltpu.make_async_copy(
        vout0.at[pl.ds(0, _ROWD * _CPS)],
        out_hbm.at[pl.ds(0, _ROWD * _CPS)], so0).wait()
    pltpu.make_async_copy(
        vout1.at[pl.ds(0, _ROWD * _CPS)],
        out_hbm.at[pl.ds(0, _ROWD * _CPS)], so1).wait()


def _sc_body(idx_hbm, t_hbm, emb_hbm, fc_hbm, lw_hbm, bias_hbm,
             out_hbm,
             idx_v, rows_v, fc_v, t_v, lw_v, bias_v, oo_v,
             sem_e, sem_f):
    nc = 2
    wid = lax.axis_index("s") * nc + lax.axis_index("c")

    pltpu.sync_copy(lw_hbm, lw_v)
    pltpu.sync_copy(bias_hbm, bias_v)
    lwv = lw_v[0, :]
    bv = bias_v[0, :]
    lanes = lax.iota(jnp.int32, 16)

    for i in range(_NCHUNK):
        b0 = wid * _BPW + i * _C
        i0 = b0 * _NF
        pltpu.sync_copy(idx_hbm.at[pl.ds(i0, _CI)], idx_v)
        pltpu.sync_copy(t_hbm.at[pl.ds(b0, _C), :], t_v)
        cp_e = pltpu.async_copy(emb_hbm.at[idx_v], rows_v, sem_e)
        cp_f = pltpu.async_copy(fc_hbm.at[idx_v], fc_v, sem_f)
        cp_e.wait()
        cp_f.wait()

        for g in range(_C // 16):
            def row_body(j, acc):
                c = g * 16 + j
                tb = t_v[c, :]
                s = tb
                q = tb * tb
                base = c * _NF
                for f in range(_NF):
                    v = rows_v[base + f, :]
                    s = s + v
                    q = q + v * v
                w = 0.5 * (s * s - q) + tb * lwv
                r = jnp.sum(w)
                return jnp.where(lanes == j, r, acc)

            fm = lax.fori_loop(0, 16, row_body, jnp.zeros((16,), jnp.float32))
            fbase = (g * 16 + lanes) * _NF
            fca = plsc.load_gather(fc_v, [fbase])
            for f in range(1, _NF):
                fca = fca + plsc.load_gather(fc_v, [fbase + f])
            oo_v[pl.ds(g * 16, 16)] = fm + fca + bv

        pltpu.sync_copy(oo_v, out_hbm.at[pl.ds(b0, _C)])


@functools.partial(jax.jit, static_argnames=())
def kernel(x, t, emb_table, fc_table, fc_bias, lin_w, lin_b):
    v4 = emb_table.reshape(_VB, 128, 2, 8).transpose(2, 0, 3, 1)
    mesh_t = plsc.VectorSubcoreMesh(core_axis_name="c", subcore_axis_name="s")
    emb_flat = pl.kernel(
        _sc_transpose_body,
        out_type=jax.ShapeDtypeStruct((_VOCAB * _D,), jnp.float32),
        mesh=mesh_t,
        compiler_params=pltpu.CompilerParams(
            needs_layout_passes=False, use_tc_tiling_on_sc=False),
        scratch_types=[
            pltpu.VMEM((2, _CPS, 8, 128), jnp.float32),
            pltpu.VMEM((2, _CPS, 8, 128), jnp.float32),
            pltpu.VMEM((_ROWS * _D + 16,), jnp.float32),
            pltpu.VMEM((_ROWS * _D + 16,), jnp.float32),
            pltpu.SemaphoreType.DMA,
            pltpu.SemaphoreType.DMA,
            pltpu.SemaphoreType.DMA,
            pltpu.SemaphoreType.DMA,
        ],
    )(v4)
    emb_rm = emb_flat.reshape(_VOCAB, _D)

    xi = x + jnp.asarray(_OFFSETS, dtype=x.dtype)[None, :]
    idx_b = xi.reshape(_B * _NF)
    bias_vec = jnp.broadcast_to((fc_bias + lin_b).reshape(1, 1), (1, _D))

    mesh = plsc.VectorSubcoreMesh(core_axis_name="c", subcore_axis_name="s")
    fn = pl.kernel(
        _sc_body,
        out_type=jax.ShapeDtypeStruct((_B,), jnp.float32),
        mesh=mesh,
        compiler_params=pltpu.CompilerParams(
            needs_layout_passes=False, use_tc_tiling_on_sc=False),
        scratch_types=[
            pltpu.VMEM((_CI,), jnp.int32),
            pltpu.VMEM((_CI, _D), jnp.float32),
            pltpu.VMEM((_CI,), jnp.float32),
            pltpu.VMEM((_C, _D), jnp.float32),
            pltpu.VMEM((1, _D), jnp.float32),
            pltpu.VMEM((1, _D), jnp.float32),
            pltpu.VMEM((_C,), jnp.float32),
            pltpu.SemaphoreType.DMA,
            pltpu.SemaphoreType.DMA,
        ],
    )
    return fn(idx_b, t, emb_rm, jnp.squeeze(fc_table, 1), lin_w, bias_vec)


# speedup vs baseline: 3.9406x; 1.0282x over previous
import functools

import jax
import jax.numpy as jnp
import numpy as np
from jax import lax
from jax.experimental import pallas as pl
from jax.experimental.pallas import tpu as pltpu
from jax.experimental.pallas import tpu_sc as plsc

_FIELD_DIMS = np.full(27, 40000, dtype=np.int64)
_USED = _FIELD_DIMS[:-1]
_NF = len(_USED)
_D = 16
_VOCAB = int(_USED.sum())
_OFFSETS = np.concatenate([[0], np.cumsum(_USED)[:-1]]).astype(np.int32)
_B = 16384

_NW = 32
_BPW = _B // _NW
_C = 64
_NCHUNK = _BPW // _C
_CI = _C * _NF

_VB = _VOCAB // 128
_CB = 125


_NCB = (_VB + _NW - 1) // _NW
_CPS = 8
_NCHT = _NCB // _CPS
_ROWS = _CPS * 128
_ROWD = 128 * _D


def _sc_transpose_body(v4_hbm, out_hbm,
                       vin0, vin1, vout0, vout1,
                       si0, si1, so0, so1):
    nc = 2
    wid = lax.axis_index("s") * nc + lax.axis_index("c")
    wc0 = wid * _NCB
    lanes16 = lax.iota(jnp.int32, 16) * 16
    idxs = [lanes16 + d for d in range(16)]
    vins = (vin0, vin1)
    vouts = (vout0, vout1)
    sis = (si0, si1)
    sos = (so0, so1)
    nch = _NCHT + 1

    def c0_of(j):
        return jnp.minimum(wc0 + j * _CPS, _VB - _CPS)

    pltpu.async_copy(v4_hbm.at[:, pl.ds(c0_of(0), _CPS), :, :], vin0, si0)
    pltpu.async_copy(v4_hbm.at[:, pl.ds(c0_of(1), _CPS), :, :], vin1, si1)

    def step(k, _):
        for par in range(2):
            j = 2 * k + par
            c0 = c0_of(j)
            vin = vins[par]
            vout = vouts[par]
            pltpu.make_async_copy(
                v4_hbm.at[:, pl.ds(c0, _CPS), :, :], vin, sis[par]).wait()

            @pl.when(k > 0)
            def _():
                pltpu.make_async_copy(
                    vout.at[pl.ds(0, _ROWD * _CPS)],
                    out_hbm.at[pl.ds(c0 * _ROWD, _ROWD * _CPS)],
                    sos[par]).wait()

            for cc in range(_CPS):
                for w0 in range(8):
                    vecs = [vin[d // 8, cc, d % 8, pl.ds(w0 * 16, 16)]
                            for d in range(16)]
                    off8 = cc * 2048 + w0 * 256
                    for d in range(16):
                        plsc.store_scatter(vout.at[pl.ds(off8, 256)],
                                           [idxs[d]], vecs[d])

            pltpu.async_copy(
                vout.at[pl.ds(0, _ROWD * _CPS)],
                out_hbm.at[pl.ds(c0 * _ROWD, _ROWD * _CPS)], sos[par])

            @pl.when(j + 2 < nch)
            def _():
                pltpu.async_copy(
                    v4_hbm.at[:, pl.ds(c0_of(j + 2), _CPS), :, :],
                    vin, sis[par])

        return 0

    lax.fori_loop(0, nch // 2, step, 0)
    pltpu.make_async_copy(
        vout0.at[pl.ds(0, _ROWD * _CPS)],
        out_hbm.at[pl.ds(0, _ROWD * _CPS)], so0).wait()
    pltpu.make_async_copy(
        vout1.at[pl.ds(0, _ROWD * _CPS)],
        out_hbm.at[pl.ds(0, _ROWD * _CPS)], so1).wait()


def _sc_body(idx_hbm, t_hbm, emb_hbm, fc_hbm, lw_hbm, bias_hbm,
             out_hbm,
             idx0, idx1, rows0, rows1, fcv0, fcv1, tv0, tv1, oo0, oo1,
             wbuf, lw_v, bias_v,
             se0, se1, sf0, sf1, so0, so1):
    nc = 2
    wid = lax.axis_index("s") * nc + lax.axis_index("c")

    pltpu.sync_copy(lw_hbm, lw_v)
    pltpu.sync_copy(bias_hbm, bias_v)
    lwv = lw_v[0, :]
    bv = bias_v[0, :]
    lanes = lax.iota(jnp.int32, 16)
    lanes16 = lanes * 16
    idxs2 = [lanes16 + d for d in range(16)]
    idxv = (idx0, idx1)
    rowsv = (rows0, rows1)
    fcv = (fcv0, fcv1)
    tv = (tv0, tv1)
    oov = (oo0, oo1)
    ses = (se0, se1)
    sfs = (sf0, sf1)
    sos = (so0, so1)

    def issue(i, par):
        b0 = wid * _BPW + i * _C
        pltpu.sync_copy(idx_hbm.at[pl.ds(b0 * _NF, _CI)], idxv[par])
        pltpu.sync_copy(t_hbm.at[pl.ds(b0, _C), :], tv[par])
        pltpu.async_copy(emb_hbm.at[idxv[par]], rowsv[par], ses[par])
        pltpu.async_copy(fc_hbm.at[idxv[par]], fcv[par], sfs[par])

    issue(0, 0)
    for i in range(_NCHUNK):
        par = i % 2
        b0 = wid * _BPW + i * _C
        rv = rowsv[par]
        fv = fcv[par]
        tb_v = tv[par]
        pltpu.make_async_copy(emb_hbm.at[idxv[par]], rv, ses[par]).wait()
        pltpu.make_async_copy(fc_hbm.at[idxv[par]], fv, sfs[par]).wait()
        if i + 1 < _NCHUNK:
            issue(i + 1, 1 - par)
        if i >= 2:
            pltpu.make_async_copy(
                oov[par], out_hbm.at[pl.ds(b0, _C)], sos[par]).wait()

        for g in range(_C // 16):
            def row_body(j, _):
                c = g * 16 + j
                tb = tb_v[c, :]
                s = tb
                q = tb * tb
                base = c * _NF
                for f in range(_NF):
                    v = rv[base + f, :]
                    s = s + v
                    q = q + v * v
                w = 0.5 * (s * s - q) + tb * lwv
                wbuf[pl.ds(j * 16, 16)] = w
                return 0

            lax.fori_loop(0, 16, row_body, 0)
            acc = plsc.load_gather(wbuf, [idxs2[0]])
            for d in range(1, 16):
                acc = acc + plsc.load_gather(wbuf, [idxs2[d]])
            fbase = (g * 16 + lanes) * _NF
            for f in range(_NF):
                acc = acc + plsc.load_gather(fv, [fbase + f])
            oov[par][pl.ds(g * 16, 16)] = acc + bv

        pltpu.async_copy(oov[par], out_hbm.at[pl.ds(b0, _C)], sos[par])

    for par in range(2):
        pltpu.make_async_copy(
            oov[par], out_hbm.at[pl.ds(0, _C)], sos[par]).wait()


@functools.partial(jax.jit, static_argnames=())
def kernel(x, t, emb_table, fc_table, fc_bias, lin_w, lin_b):
    v4 = emb_table.reshape(_VB, 128, 2, 8).transpose(2, 0, 3, 1)
    mesh_t = plsc.VectorSubcoreMesh(core_axis_name="c", subcore_axis_name="s")
    emb_flat = pl.kernel(
        _sc_transpose_body,
        out_type=jax.ShapeDtypeStruct((_VOCAB * _D,), jnp.float32),
        mesh=mesh_t,
        compiler_params=pltpu.CompilerParams(
            needs_layout_passes=False, use_tc_tiling_on_sc=False),
        scratch_types=[
            pltpu.VMEM((2, _CPS, 8, 128), jnp.float32),
            pltpu.VMEM((2, _CPS, 8, 128), jnp.float32),
            pltpu.VMEM((_ROWS * _D + 16,), jnp.float32),
            pltpu.VMEM((_ROWS * _D + 16,), jnp.float32),
            pltpu.SemaphoreType.DMA,
            pltpu.SemaphoreType.DMA,
            pltpu.SemaphoreType.DMA,
            pltpu.SemaphoreType.DMA,
        ],
    )(v4)
    emb_rm = emb_flat.reshape(_VOCAB, _D)

    xi = x + jnp.asarray(_OFFSETS, dtype=x.dtype)[None, :]
    idx_b = xi.reshape(_B * _NF)
    bias_vec = jnp.broadcast_to((fc_bias + lin_b).reshape(1, 1), (1, _D))

    mesh = plsc.VectorSubcoreMesh(core_axis_name="c", subcore_axis_name="s")
    fn = pl.kernel(
        _sc_body,
        out_type=jax.ShapeDtypeStruct((_B,), jnp.float32),
        mesh=mesh,
        compiler_params=pltpu.CompilerParams(
            needs_layout_passes=False, use_tc_tiling_on_sc=False),
        scratch_types=[
            pltpu.VMEM((_CI,), jnp.int32),
            pltpu.VMEM((_CI,), jnp.int32),
            pltpu.VMEM((_CI, _D), jnp.float32),
            pltpu.VMEM((_CI, _D), jnp.float32),
            pltpu.VMEM((_CI,), jnp.float32),
            pltpu.VMEM((_CI,), jnp.float32),
            pltpu.VMEM((_C, _D), jnp.float32),
            pltpu.VMEM((_C, _D), jnp.float32),
            pltpu.VMEM((_C,), jnp.float32),
            pltpu.VMEM((_C,), jnp.float32),
            pltpu.VMEM((256,), jnp.float32),
            pltpu.VMEM((1, _D), jnp.float32),
            pltpu.VMEM((1, _D), jnp.float32),
            pltpu.SemaphoreType.DMA,
            pltpu.SemaphoreType.DMA,
            pltpu.SemaphoreType.DMA,
            pltpu.SemaphoreType.DMA,
            pltpu.SemaphoreType.DMA,
            pltpu.SemaphoreType.DMA,
        ],
    )
    return fn(idx_b, t, emb_rm, jnp.squeeze(fc_table, 1), lin_w, bias_vec)


# speedup vs baseline: 5.1903x; 1.3172x over previous
import functools

import jax
import jax.numpy as jnp
import numpy as np
from jax import lax
from jax.experimental import pallas as pl
from jax.experimental.pallas import tpu as pltpu
from jax.experimental.pallas import tpu_sc as plsc

_FIELD_DIMS = np.full(27, 40000, dtype=np.int64)
_USED = _FIELD_DIMS[:-1]
_NF = len(_USED)
_D = 16
_VOCAB = int(_USED.sum())
_OFFSETS = np.concatenate([[0], np.cumsum(_USED)[:-1]]).astype(np.int32)
_B = 16384

_NW = 32
_BPW = _B // _NW
_C = 64
_NCHUNK = _BPW // _C
_CI = _C * _NF

_VB = _VOCAB // 128
_CB = 125


_NCB = (_VB + _NW - 1) // _NW
_CPS = 8
_NCHT = _NCB // _CPS
_ROWS = _CPS * 128
_ROWD = 128 * _D


def _sc_transpose_body(v4_hbm, out_hbm,
                       vin0, vin1, vout0, vout1,
                       si0, si1, so0, so1):
    nc = 2
    wid = lax.axis_index("s") * nc + lax.axis_index("c")
    wc0 = wid * _NCB
    lanes16 = lax.iota(jnp.int32, 16) * 16
    idxs8 = [lanes16 + d for d in range(8)]
    vins = (vin0, vin1)
    vouts = (vout0, vout1)
    sis = (si0, si1)
    sos = (so0, so1)
    nch = _NCHT + 1

    def c0_of(j):
        return jnp.minimum(wc0 + j * _CPS, _VB - _CPS)

    pltpu.async_copy(v4_hbm.at[:, pl.ds(c0_of(0), _CPS), :, :], vin0, si0)
    pltpu.async_copy(v4_hbm.at[:, pl.ds(c0_of(1), _CPS), :, :], vin1, si1)

    def step(k, _):
        for par in range(2):
            j = 2 * k + par
            c0 = c0_of(j)
            vin = vins[par]
            vout = vouts[par]
            pltpu.make_async_copy(
                v4_hbm.at[:, pl.ds(c0, _CPS), :, :], vin, sis[par]).wait()

            @pl.when(k > 0)
            def _():
                pltpu.make_async_copy(
                    vout.at[pl.ds(0, _ROWD * _CPS)],
                    out_hbm.at[pl.ds(c0 * _ROWD, _ROWD * _CPS)],
                    sos[par]).wait()

            def ld(cc, w0, d):
                return vin[d // 8, cc, d % 8, pl.ds(w0 * 16, 16)]

            def sc(cc, w0, d, vec):
                off = cc * 2048 + w0 * 256 + 8 * (d // 8)
                plsc.store_scatter(vout.at[pl.ds(off, 256)],
                                   [idxs8[d % 8]], vec)

            groups = [(cc, w0) for cc in range(_CPS) for w0 in range(8)]
            prev = [ld(*groups[0], d) for d in range(16)]
            pg = groups[0]
            for gi in range(1, len(groups)):
                cur = []
                for d in range(16):
                    sc(pg[0], pg[1], d, prev[d])
                    cur.append(ld(*groups[gi], d))
                prev, pg = cur, groups[gi]
            for d in range(16):
                sc(pg[0], pg[1], d, prev[d])

            pltpu.async_copy(
                vout.at[pl.ds(0, _ROWD * _CPS)],
                out_hbm.at[pl.ds(c0 * _ROWD, _ROWD * _CPS)], sos[par])

            @pl.when(j + 2 < nch)
            def _():
                pltpu.async_copy(
                    v4_hbm.at[:, pl.ds(c0_of(j + 2), _CPS), :, :],
                    vin, sis[par])

        return 0

    lax.fori_loop(0, nch // 2, step, 0)
    pltpu.make_async_copy(
        vout0.at[pl.ds(0, _ROWD * _CPS)],
        out_hbm.at[pl.ds(0, _ROWD * _CPS)], so0).wait()
    pltpu.make_async_copy(
        vout1.at[pl.ds(0, _ROWD * _CPS)],
        out_hbm.at[pl.ds(0, _ROWD * _CPS)], so1).wait()


def _sc_body(idx_hbm, t_hbm, emb_hbm, fc_hbm, lw_hbm, bias_hbm,
             out_hbm,
             idx0, idx1, rows0, rows1, fcv0, fcv1, tv0, tv1, oo0, oo1,
             wbuf, lw_v, bias_v,
             se0, se1, sf0, sf1, so0, so1):
    nc = 2
    wid = lax.axis_index("s") * nc + lax.axis_index("c")

    pltpu.sync_copy(lw_hbm, lw_v)
    pltpu.sync_copy(bias_hbm, bias_v)
    lwv = lw_v[0, :]
    bv = bias_v[0, :]
    lanes = lax.iota(jnp.int32, 16)
    lanes16 = lanes * 16
    idxs2 = [lanes16 + d for d in range(16)]
    idxv = (idx0, idx1)
    rowsv = (rows0, rows1)
    fcv = (fcv0, fcv1)
    tv = (tv0, tv1)
    oov = (oo0, oo1)
    ses = (se0, se1)
    sfs = (sf0, sf1)
    sos = (so0, so1)

    def issue(i, par):
        b0 = wid * _BPW + i * _C
        pltpu.sync_copy(idx_hbm.at[pl.ds(b0 * _NF, _CI)], idxv[par])
        pltpu.sync_copy(t_hbm.at[pl.ds(b0, _C), :], tv[par])
        pltpu.async_copy(emb_hbm.at[idxv[par]], rowsv[par], ses[par])
        pltpu.async_copy(fc_hbm.at[idxv[par]], fcv[par], sfs[par])

    issue(0, 0)
    for i in range(_NCHUNK):
        par = i % 2
        b0 = wid * _BPW + i * _C
        rv = rowsv[par]
        fv = fcv[par]
        tb_v = tv[par]
        pltpu.make_async_copy(emb_hbm.at[idxv[par]], rv, ses[par]).wait()
        pltpu.make_async_copy(fc_hbm.at[idxv[par]], fv, sfs[par]).wait()
        if i + 1 < _NCHUNK:
            issue(i + 1, 1 - par)
        if i >= 2:
            pltpu.make_async_copy(
                oov[par], out_hbm.at[pl.ds(b0, _C)], sos[par]).wait()

        for g in range(_C // 16):
            def row_body(j, _):
                c = g * 16 + j
                tb = tb_v[c, :]
                s = tb
                q = tb * tb
                base = c * _NF
                for f in range(_NF):
                    v = rv[base + f, :]
                    s = s + v
                    q = q + v * v
                w = 0.5 * (s * s - q) + tb * lwv
                wbuf[pl.ds(j * 16, 16)] = w
                return 0

            lax.fori_loop(0, 16, row_body, 0)
            acc = plsc.load_gather(wbuf, [idxs2[0]])
            for d in range(1, 16):
                acc = acc + plsc.load_gather(wbuf, [idxs2[d]])
            fbase = (g * 16 + lanes) * _NF
            for f in range(_NF):
                acc = acc + plsc.load_gather(fv, [fbase + f])
            oov[par][pl.ds(g * 16, 16)] = acc + bv

        pltpu.async_copy(oov[par], out_hbm.at[pl.ds(b0, _C)], sos[par])

    for par in range(2):
        pltpu.make_async_copy(
            oov[par], out_hbm.at[pl.ds(0, _C)], sos[par]).wait()


@functools.partial(jax.jit, static_argnames=())
def kernel(x, t, emb_table, fc_table, fc_bias, lin_w, lin_b):
    v4 = emb_table.reshape(_VB, 128, 2, 8).transpose(2, 0, 3, 1)
    mesh_t = plsc.VectorSubcoreMesh(core_axis_name="c", subcore_axis_name="s")
    emb_flat = pl.kernel(
        _sc_transpose_body,
        out_type=jax.ShapeDtypeStruct((_VOCAB * _D,), jnp.float32),
        mesh=mesh_t,
        compiler_params=pltpu.CompilerParams(
            needs_layout_passes=False, use_tc_tiling_on_sc=False),
        scratch_types=[
            pltpu.VMEM((2, _CPS, 8, 128), jnp.float32),
            pltpu.VMEM((2, _CPS, 8, 128), jnp.float32),
            pltpu.VMEM((_ROWS * _D + 16,), jnp.float32),
            pltpu.VMEM((_ROWS * _D + 16,), jnp.float32),
            pltpu.SemaphoreType.DMA,
            pltpu.SemaphoreType.DMA,
            pltpu.SemaphoreType.DMA,
            pltpu.SemaphoreType.DMA,
        ],
    )(v4)
    emb_rm = emb_flat.reshape(_VOCAB, _D)

    xi = x + jnp.asarray(_OFFSETS, dtype=x.dtype)[None, :]
    idx_b = xi.reshape(_B * _NF)
    bias_vec = jnp.broadcast_to((fc_bias + lin_b).reshape(1, 1), (1, _D))

    mesh = plsc.VectorSubcoreMesh(core_axis_name="c", subcore_axis_name="s")
    fn = pl.kernel(
        _sc_body,
        out_type=jax.ShapeDtypeStruct((_B,), jnp.float32),
        mesh=mesh,
        compiler_params=pltpu.CompilerParams(
            needs_layout_passes=False, use_tc_tiling_on_sc=False),
        scratch_types=[
            pltpu.VMEM((_CI,), jnp.int32),
            pltpu.VMEM((_CI,), jnp.int32),
            pltpu.VMEM((_CI, _D), jnp.float32),
            pltpu.VMEM((_CI, _D), jnp.float32),
            pltpu.VMEM((_CI,), jnp.float32),
            pltpu.VMEM((_CI,), jnp.float32),
            pltpu.VMEM((_C, _D), jnp.float32),
            pltpu.VMEM((_C, _D), jnp.float32),
            pltpu.VMEM((_C,), jnp.float32),
            pltpu.VMEM((_C,), jnp.float32),
            pltpu.VMEM((256,), jnp.float32),
            pltpu.VMEM((1, _D), jnp.float32),
            pltpu.VMEM((1, _D), jnp.float32),
            pltpu.SemaphoreType.DMA,
            pltpu.SemaphoreType.DMA,
            pltpu.SemaphoreType.DMA,
            pltpu.SemaphoreType.DMA,
            pltpu.SemaphoreType.DMA,
            pltpu.SemaphoreType.DMA,
        ],
    )
    return fn(idx_b, t, emb_rm, jnp.squeeze(fc_table, 1), lin_w, bias_vec)
